# baseline (speedup 1.0000x reference)
import functools

import jax
import jax.numpy as jnp
from jax import lax
from jax.experimental import pallas as pl
from jax.experimental.pallas import tpu as pltpu
from jax.experimental.pallas import tpu_sc as plsc

NUM_FIELDS = 26
VOCAB = 100000
EMBED_DIM = 16
BATCH = 16384

N_JOBS = NUM_FIELDS * EMBED_DIM
NW = 32
JOBS_PER_W = N_JOBS // NW
CHUNK = 4096
N_CHUNKS = BATCH // CHUNK
UNROLL = 32


def _build_sc_gather():
    mesh = plsc.VectorSubcoreMesh(core_axis_name="c", subcore_axis_name="s")

    @functools.partial(
        pl.kernel,
        out_type=jax.ShapeDtypeStruct((N_JOBS, BATCH), jnp.float32),
        mesh=mesh,
        scratch_types=[
            pltpu.VMEM((VOCAB,), jnp.float32),
            pltpu.VMEM((BATCH,), jnp.int32),
            pltpu.VMEM((2, CHUNK), jnp.float32),
            pltpu.SemaphoreType.DMA,
        ],
        compiler_params=pltpu.CompilerParams(use_tc_tiling_on_sc=True,
                                             needs_layout_passes=False),
    )
    def gather_kernel(xt_hbm, tab_hbm, out_hbm, slab_v, col_v, obuf_v,
                      w_sem):
        wid = lax.axis_index("s") * 2 + lax.axis_index("c")
        j0 = wid * JOBS_PER_W

        def do_job(t, carry):
            j = j0 + t
            f = j // EMBED_DIM
            pltpu.sync_copy(tab_hbm.at[f, j - f * EMBED_DIM], slab_v)

            @pl.when(jnp.logical_or(t == 0, f * EMBED_DIM == j))
            def _():
                pltpu.sync_copy(xt_hbm.at[f], col_v)

            for k in range(N_CHUNKS):
                half = k % 2
                if k >= 2:
                    pltpu.make_async_copy(
                        obuf_v.at[half],
                        out_hbm.at[j0, pl.ds(0, CHUNK)], w_sem).wait()
                elif k < 2:
                    @pl.when(t > 0)
                    def _():
                        pltpu.make_async_copy(
                            obuf_v.at[half],
                            out_hbm.at[j0, pl.ds(0, CHUNK)], w_sem).wait()

                def gath(p, c2):
                    base = k * CHUNK + p * (16 * UNROLL)
                    for u in range(UNROLL):
                        sl = pl.ds(base + u * 16, 16)
                        osl = pl.ds(base + u * 16 - k * CHUNK, 16)
                        idx = col_v[sl]
                        obuf_v[half, osl] = plsc.load_gather(slab_v, [idx])
                    return c2

                lax.fori_loop(0, CHUNK // (16 * UNROLL), gath, 0)
                pltpu.async_copy(obuf_v.at[half],
                                 out_hbm.at[j, pl.ds(k * CHUNK, CHUNK)],
                                 w_sem)
            return carry

        lax.fori_loop(0, JOBS_PER_W, do_job, 0)
        for _ in range(2):
            pltpu.make_async_copy(obuf_v.at[0],
                                  out_hbm.at[j0, pl.ds(0, CHUNK)],
                                  w_sem).wait()

    return gather_kernel


_sc_gather = _build_sc_gather()


@jax.jit
def kernel(x, tables):
    xt = x.astype(jnp.int32).T
    tabt = tables.transpose(0, 2, 1)
    out = _sc_gather(xt, tabt)
    return out.reshape(NUM_FIELDS, EMBED_DIM, BATCH).transpose(2, 0, 1)
